# baseline (speedup 1.0000x reference)
import functools

import jax
import jax.numpy as jnp
from jax import lax
from jax.experimental import pallas as pl
from jax.experimental.pallas import tpu as pltpu
from jax.experimental.pallas import tpu_sc as plsc

N = 10000
E = 320000
D = 128

NC = 2
NS = 16
CHUNK = 64
CHUNKS_PER_TILE = 160
PASS_CHUNKS = 40
NPASS = CHUNKS_PER_TILE // PASS_CHUNKS
EP = NC * NS * CHUNKS_PER_TILE * CHUNK
EROWS = EP // CHUNK
N_PAD = 10240
ROWS_PER_TILE = N_PAD // NS
NZCOPY = ROWS_PER_TILE // CHUNK


def _sc_body(x_hbm, src_hbm, dst_hbm, w_hbm, out_hbm,
             esrc, edst, ew, b0, b1, b2, b3, agg, g0, g1, g2, g3):
    c = lax.axis_index("c")
    s = lax.axis_index("s")
    brows = (b0, b1, b2, b3)
    gsem = (g0, g1, g2, g3)

    def _zrow(i, _):
        for f in range(D // 16):
            b0[i, pl.ds(f * 16, 16)] = jnp.zeros((16,), jnp.float32)
        return 0
    lax.fori_loop(0, CHUNK, _zrow, 0)
    zbase = s * ROWS_PER_TILE
    for q in range(NZCOPY):
        pltpu.sync_copy(b0, agg.at[pl.ds(zbase + q * CHUNK, CHUNK)])
    plsc.subcore_barrier()

    tile_row0 = (c * NS + s) * CHUNKS_PER_TILE

    def _gather(ch, u):
        pltpu.async_copy(x_hbm.at[esrc.at[ch]], brows[u % 4], gsem[u % 4])

    def _gwait(ch, u):
        pltpu.make_async_copy(x_hbm.at[esrc.at[ch]], brows[u % 4],
                              gsem[u % 4]).wait()

    def _scale(ch, u):
        buf = brows[u % 4]

        def g_body(g, _):
            wv16 = ew[ch, pl.ds(g * 16, 16)]
            for e in range(16):
                i = g * 16 + e
                wv = jnp.full((16,), wv16[e], jnp.float32)
                for f in range(D // 16):
                    sl = pl.ds(f * 16, 16)
                    buf[i, sl] = buf[i, sl] * wv
            return 0
        lax.fori_loop(0, CHUNK // 16, g_body, 0)

    def _pass(p, _):
        prow = tile_row0 + p * PASS_CHUNKS
        pltpu.sync_copy(src_hbm.at[pl.ds(prow, PASS_CHUNKS)], esrc)
        pltpu.sync_copy(dst_hbm.at[pl.ds(prow, PASS_CHUNKS)], edst)
        pltpu.sync_copy(w_hbm.at[pl.ds(prow, PASS_CHUNKS)], ew)

        _gather(0, 0)
        _gather(1, 1)
        _gather(2, 2)

        def _quad(j, _):
            for u in range(4):
                ch = 4 * j + u
                if u == 0:
                    _gather(ch + 3, u + 3)
                else:
                    @pl.when(j < PASS_CHUNKS // 4 - 1)
                    def _():
                        _gather(ch + 3, u + 3)
                _gwait(ch, u)
                _scale(ch, u)
                pltpu.sync_copy(brows[u], agg.at[edst.at[ch]], add=True)
            return 0
        lax.fori_loop(0, PASS_CHUNKS // 4, _quad, 0)
        return 0
    lax.fori_loop(0, NPASS, _pass, 0)
    plsc.subcore_barrier()

    for q in range(NZCOPY):
        o = zbase + q * CHUNK
        pltpu.sync_copy(agg.at[pl.ds(o, CHUNK)], b0)
        pltpu.sync_copy(b0, out_hbm.at[c, pl.ds(o, CHUNK)])


_sc_agg = functools.partial(
    pl.kernel,
    out_type=jax.ShapeDtypeStruct((NC, N_PAD, D), jnp.float32),
    mesh=plsc.VectorSubcoreMesh(core_axis_name="c", subcore_axis_name="s",
                                num_cores=NC, num_subcores=NS),
    scratch_types=[
        pltpu.VMEM((PASS_CHUNKS, CHUNK), jnp.int32),
        pltpu.VMEM((PASS_CHUNKS, CHUNK), jnp.int32),
        pltpu.VMEM((PASS_CHUNKS, CHUNK), jnp.float32),
        pltpu.VMEM((CHUNK, D), jnp.float32),
        pltpu.VMEM((CHUNK, D), jnp.float32),
        pltpu.VMEM((CHUNK, D), jnp.float32),
        pltpu.VMEM((CHUNK, D), jnp.float32),
        pltpu.VMEM_SHARED((N_PAD, D), jnp.float32),
        pltpu.SemaphoreType.DMA,
        pltpu.SemaphoreType.DMA,
        pltpu.SemaphoreType.DMA,
        pltpu.SemaphoreType.DMA,
    ],
)(_sc_body)


def _tc_body(scal_ref, p_ref, h_ref, w_ref, o_ref):
    alpha = scal_ref[0]
    beta = scal_ref[1]
    left = (1.0 - alpha) * (p_ref[0] + p_ref[1]) + alpha * h_ref[...]
    o_ref[...] = (1.0 - beta) * left + beta * jnp.dot(
        left, w_ref[...], preferred_element_type=jnp.float32)


_ROWS_BLK = 1000

_tc_finish = pl.pallas_call(
    _tc_body,
    grid=(N // _ROWS_BLK,),
    in_specs=[
        pl.BlockSpec(memory_space=pltpu.SMEM),
        pl.BlockSpec((NC, _ROWS_BLK, D), lambda i: (0, i, 0)),
        pl.BlockSpec((_ROWS_BLK, D), lambda i: (i, 0)),
        pl.BlockSpec((D, D), lambda i: (0, 0)),
    ],
    out_specs=pl.BlockSpec((_ROWS_BLK, D), lambda i: (i, 0)),
    out_shape=jax.ShapeDtypeStruct((N, D), jnp.float32),
)


def kernel(x, h0, W, adj_values, adj_edge_index, alpha, beta):
    dst = adj_edge_index[0]
    src = adj_edge_index[1]
    pad = EP - E
    srcp = jnp.concatenate([src, jnp.zeros((pad,), src.dtype)]).reshape(
        EROWS, CHUNK)
    dstp = jnp.concatenate([dst, jnp.zeros((pad,), dst.dtype)]).reshape(
        EROWS, CHUNK)
    wp = jnp.concatenate(
        [adj_values, jnp.zeros((pad,), adj_values.dtype)]).reshape(
        EROWS, CHUNK)
    partials = _sc_agg(x, srcp, dstp, wp)
    scal = jnp.stack([jnp.asarray(alpha, jnp.float32),
                      jnp.asarray(beta, jnp.float32)])
    return _tc_finish(scal, partials, h0, W)
